# baseline (speedup 1.0000x reference)
import dataclasses
import functools

import jax
import jax.numpy as jnp
from jax import lax
from jax.experimental import pallas as pl
from jax.experimental.pallas import tpu as pltpu
from jax.experimental.pallas import tpu_sc as plsc

NUM_GRAPHS = 4096
TOTAL_ATOMS = 524288
NLANES = 16
NC, NS = 2, 16
NW = NC * NS
CHUNK = TOTAL_ATOMS // NW


def _shift_gather(x, idx):
    dnums = lax.GatherDimensionNumbers(
        offset_dims=(), collapsed_slice_dims=(0,), start_index_map=(0,))
    return lax.gather(x, idx[:, None], dnums, slice_sizes=(1,),
                      mode=lax.GatherScatterMode.PROMISE_IN_BOUNDS)


def _sc_partials(z_hbm, g_hbm, w_hbm, sums_out, cnts_out,
                 zb, gb, wv, sums, cnts, sem_z, sem_g, sem_w):
    wid = lax.axis_index("s") * NC + lax.axis_index("c")
    base = wid * CHUNK

    cp_z = pltpu.async_copy(z_hbm.at[pl.ds(base, CHUNK)], zb, sem_z)
    cp_g = pltpu.async_copy(g_hbm.at[pl.ds(base, CHUNK)], gb, sem_g)
    cp_w = pltpu.async_copy(w_hbm, wv, sem_w)

    zero16 = jnp.zeros((NLANES,), jnp.float32)

    @pl.loop(0, NUM_GRAPHS, step=NLANES)
    def _(j):
        sums[pl.ds(j, NLANES)] = zero16
        cnts[pl.ds(j, NLANES)] = zero16

    cp_z.wait()
    cp_g.wait()
    cp_w.wait()

    iota = lax.iota(jnp.int32, NLANES)

    @plsc.parallel_loop(0, CHUNK, step=NLANES, unroll=4)
    def _(i):
        g = gb[pl.ds(i, NLANES)]
        z = zb[pl.ds(i, NLANES)]
        w = plsc.load_gather(wv, [z])
        cnt, rend = plsc.scan_count(g)
        cum = plsc.cumsum(w)
        pe = iota - cnt
        bsv = jnp.where(pe >= 0, _shift_gather(cum, jnp.maximum(pe, 0)), 0.0)
        plsc.addupdate_scatter(sums, [g], cum - bsv, mask=rend)
        plsc.addupdate_scatter(cnts, [g], cnt.astype(jnp.float32), mask=rend)

    pltpu.sync_copy(sums, sums_out.at[wid])
    pltpu.sync_copy(cnts, cnts_out.at[wid])


def _combine_body(s_ref, c_ref, o_ref):
    s = jnp.sum(s_ref[...], axis=0, keepdims=True)
    c = jnp.sum(c_ref[...], axis=0, keepdims=True)
    o_ref[...] = s / jnp.maximum(c, 1.0)


@jax.jit
def kernel(atomic_number, graph_ids, W):
    z = atomic_number.astype(jnp.int32)
    g = graph_ids.astype(jnp.int32)
    wp = W.reshape(-1).astype(jnp.float32)

    mesh = plsc.VectorSubcoreMesh(core_axis_name="c", subcore_axis_name="s")
    f32 = jnp.float32
    cp = pltpu.CompilerParams()
    if "needs_layout_passes" in pltpu.CompilerParams.__dataclass_fields__:
        cp = dataclasses.replace(cp, needs_layout_passes=False)
    sc = pl.kernel(
        _sc_partials,
        out_type=(jax.ShapeDtypeStruct((NW, NUM_GRAPHS), f32),
                  jax.ShapeDtypeStruct((NW, NUM_GRAPHS), f32)),
        mesh=mesh,
        scratch_types=[
            pltpu.VMEM((CHUNK,), jnp.int32),
            pltpu.VMEM((CHUNK,), jnp.int32),
            pltpu.VMEM((94,), f32),
            pltpu.VMEM((NUM_GRAPHS,), f32),
            pltpu.VMEM((NUM_GRAPHS,), f32),
            pltpu.SemaphoreType.DMA,
            pltpu.SemaphoreType.DMA,
            pltpu.SemaphoreType.DMA,
        ],
        compiler_params=cp,
    )
    sums, cnts = sc(z, g, wp)

    energy = pl.pallas_call(
        _combine_body,
        out_shape=jax.ShapeDtypeStruct((1, NUM_GRAPHS), f32),
    )(sums, cnts)
    return energy.reshape(-1)


# speedup vs baseline: 1.0063x; 1.0063x over previous
import dataclasses
import functools

import jax
import jax.numpy as jnp
from jax import lax
from jax.experimental import pallas as pl
from jax.experimental.pallas import tpu as pltpu
from jax.experimental.pallas import tpu_sc as plsc

NUM_GRAPHS = 4096
TOTAL_ATOMS = 524288
NLANES = 16
NC, NS = 2, 16
NW = NC * NS
CHUNK = TOTAL_ATOMS // NW


def _shift_gather(x, idx):
    dnums = lax.GatherDimensionNumbers(
        offset_dims=(), collapsed_slice_dims=(0,), start_index_map=(0,))
    return lax.gather(x, idx[:, None], dnums, slice_sizes=(1,),
                      mode=lax.GatherScatterMode.PROMISE_IN_BOUNDS)


def _sc_partials(z_hbm, g_hbm, w_hbm, sums_out, cnts_out,
                 zb, gb, wv, sums, cnts, sem_z, sem_g, sem_w):
    wid = lax.axis_index("s") * NC + lax.axis_index("c")
    base = wid * CHUNK

    half = CHUNK // 2
    cp_z0 = pltpu.async_copy(z_hbm.at[pl.ds(base, half)],
                             zb.at[pl.ds(0, half)], sem_z)
    cp_g0 = pltpu.async_copy(g_hbm.at[pl.ds(base, half)],
                             gb.at[pl.ds(0, half)], sem_g)
    cp_w = pltpu.async_copy(w_hbm, wv, sem_w)

    zero16 = jnp.zeros((NLANES,), jnp.float32)

    @pl.loop(0, NUM_GRAPHS, step=NLANES)
    def _(j):
        sums[pl.ds(j, NLANES)] = zero16
        cnts[pl.ds(j, NLANES)] = zero16

    iota = lax.iota(jnp.int32, NLANES)

    def segsum_block(lo, hi):
        @plsc.parallel_loop(lo, hi, step=NLANES, unroll=4)
        def _(i):
            g = gb[pl.ds(i, NLANES)]
            z = zb[pl.ds(i, NLANES)]
            w = plsc.load_gather(wv, [z])
            cnt, rend = plsc.scan_count(g)
            cum = plsc.cumsum(w)
            pe = iota - cnt
            bsv = jnp.where(pe >= 0,
                            _shift_gather(cum, jnp.maximum(pe, 0)), 0.0)
            plsc.addupdate_scatter(sums, [g], cum - bsv, mask=rend)
            plsc.addupdate_scatter(cnts, [g], cnt.astype(jnp.float32),
                                   mask=rend)

    cp_z0.wait()
    cp_g0.wait()
    cp_w.wait()
    cp_z1 = pltpu.async_copy(z_hbm.at[pl.ds(base + half, half)],
                             zb.at[pl.ds(half, half)], sem_z)
    cp_g1 = pltpu.async_copy(g_hbm.at[pl.ds(base + half, half)],
                             gb.at[pl.ds(half, half)], sem_g)
    segsum_block(0, half)
    cp_z1.wait()
    cp_g1.wait()
    segsum_block(half, CHUNK)

    pltpu.sync_copy(sums, sums_out.at[wid])
    pltpu.sync_copy(cnts, cnts_out.at[wid])


def _combine_body(s_ref, c_ref, o_ref):
    s = jnp.sum(s_ref[...], axis=0, keepdims=True)
    c = jnp.sum(c_ref[...], axis=0, keepdims=True)
    o_ref[...] = s / jnp.maximum(c, 1.0)


@jax.jit
def kernel(atomic_number, graph_ids, W):
    z = atomic_number.astype(jnp.int32)
    g = graph_ids.astype(jnp.int32)
    wp = W.reshape(-1).astype(jnp.float32)

    mesh = plsc.VectorSubcoreMesh(core_axis_name="c", subcore_axis_name="s")
    f32 = jnp.float32
    cp = pltpu.CompilerParams()
    if "needs_layout_passes" in pltpu.CompilerParams.__dataclass_fields__:
        cp = dataclasses.replace(cp, needs_layout_passes=False)
    sc = pl.kernel(
        _sc_partials,
        out_type=(jax.ShapeDtypeStruct((NW, NUM_GRAPHS), f32),
                  jax.ShapeDtypeStruct((NW, NUM_GRAPHS), f32)),
        mesh=mesh,
        scratch_types=[
            pltpu.VMEM((CHUNK,), jnp.int32),
            pltpu.VMEM((CHUNK,), jnp.int32),
            pltpu.VMEM((94,), f32),
            pltpu.VMEM((NUM_GRAPHS,), f32),
            pltpu.VMEM((NUM_GRAPHS,), f32),
            pltpu.SemaphoreType.DMA,
            pltpu.SemaphoreType.DMA,
            pltpu.SemaphoreType.DMA,
        ],
        compiler_params=cp,
    )
    sums, cnts = sc(z, g, wp)

    energy = pl.pallas_call(
        _combine_body,
        out_shape=jax.ShapeDtypeStruct((1, NUM_GRAPHS), f32),
    )(sums, cnts)
    return energy.reshape(-1)
